# baseline (speedup 1.0000x reference)
import numpy as np
import jax
import jax.numpy as jnp
from jax.experimental import pallas as pl
from jax.experimental.pallas import tpu as pltpu

_B, _K, _N, _M = 256, 8, 64, 32
_P = _N * (_N - 1) // 2
_R = _B * _K
_KP = 2048
_NBLK = _KP // 128

_ROWS, _COLS = np.triu_indices(_N, k=1)
_OFF = np.concatenate([[0], np.cumsum(np.arange(_N - 1, 0, -1))]).astype(np.int64)

_RT_NP = np.zeros((_N, _P), np.float32)
_RT_NP[_ROWS, np.arange(_P)] += 1.0
_RT_NP[_COLS, np.arange(_P)] += 1.0

_PIECES = []
for _bs in range(0, _KP, 128):
    _plist = []
    _be = _bs + 128
    for _i in range(_N - 1):
        _s = int(_OFF[_i])
        _e = _s + (_N - 1 - _i)
        _lo, _hi = max(_s, _bs), min(_e, _be)
        if _lo < _hi:
            _plist.append((_i, _i + 1 + (_lo - _s), _hi - _lo))
    _PIECES.append(_plist)


def _logic_kernel(x_ref, w_ref, b_ref, rt_ref, out_ref, ap_ref):
    wa = w_ref[0:_P, :]
    wo = w_ref[_P:2 * _P, :]
    wx = w_ref[2 * _P:3 * _P, :]
    ap_ref[0:_P, :] = wa - wo - 2.0 * wx
    ap_ref[_P:_KP, :] = jnp.zeros((_KP - _P, _M), jnp.float32)
    c = wo + wx
    clin = jnp.dot(rt_ref[:, :], c, preferred_element_type=jnp.float32)

    x = x_ref[:, :]
    acc0 = jnp.dot(x, clin, preferred_element_type=jnp.float32) + b_ref[:, :]
    accs = [acc0,
            jnp.zeros((_R, _M), jnp.float32),
            jnp.zeros((_R, _M), jnp.float32),
            jnp.zeros((_R, _M), jnp.float32)]
    for bi in range(_NBLK):
        parts = [x[:, j0:j0 + w] * x[:, i:i + 1] for (i, j0, w) in _PIECES[bi]]
        used = sum(w for (_, _, w) in _PIECES[bi])
        if used < 128:
            parts.append(jnp.zeros((_R, 128 - used), jnp.float32))
        lhs = jnp.concatenate(parts, axis=1)
        accs[bi % 4] = accs[bi % 4] + jnp.dot(
            lhs, ap_ref[128 * bi:128 * (bi + 1), :],
            preferred_element_type=jnp.float32)
    out_ref[:, :] = (accs[0] + accs[1]) + (accs[2] + accs[3])


def kernel(inputs, W, b):
    x2d = inputs.reshape(_R, _N)
    b2d = b.reshape(1, _M)
    rt = jnp.asarray(_RT_NP)
    out = pl.pallas_call(
        _logic_kernel,
        out_shape=jax.ShapeDtypeStruct((_R, _M), jnp.float32),
        scratch_shapes=[pltpu.VMEM((_KP, _M), jnp.float32)],
    )(x2d, W, b2d, rt)
    return out.reshape(_B, _K, _M)


# speedup vs baseline: 1.1860x; 1.1860x over previous
import numpy as np
import jax
import jax.numpy as jnp
from jax.experimental import pallas as pl
from jax.experimental.pallas import tpu as pltpu

_B, _K, _N, _M = 256, 8, 64, 32
_P = _N * (_N - 1) // 2
_R = _B * _K

_ROWS, _COLS = np.triu_indices(_N, k=1)
_OFF = np.concatenate([[0], np.cumsum(np.arange(_N - 1, 0, -1))]).astype(np.int64)

_RT_NP = np.zeros((_N, _P), np.float32)
_RT_NP[_ROWS, np.arange(_P)] += 1.0
_RT_NP[_COLS, np.arange(_P)] += 1.0


def _logic_kernel(x_ref, w_ref, b_ref, rt_ref, out_ref, w2_ref):
    wa = w_ref[0:_P, :]
    wo = w_ref[_P:2 * _P, :]
    wx = w_ref[2 * _P:3 * _P, :]
    a = (wa - wo - 2.0 * wx).astype(jnp.bfloat16)
    c = wo + wx

    w2_ref[:, :] = jnp.zeros((_N * _N, _M), jnp.bfloat16)
    for i in range(_N - 1):
        cnt = _N - 1 - i
        w2_ref[i * _N + i + 1:i * _N + _N, :] = a[int(_OFF[i]):int(_OFF[i]) + cnt, :]

    clin = jnp.dot(rt_ref[:, :], c, preferred_element_type=jnp.float32)

    x = x_ref[:, :]
    acc0 = jnp.dot(x, clin, preferred_element_type=jnp.float32) + b_ref[:, :]
    accs = [acc0,
            jnp.zeros((_R, _M), jnp.float32),
            jnp.zeros((_R, _M), jnp.float32),
            jnp.zeros((_R, _M), jnp.float32)]
    for g in range(_N // 4):
        lhs = jnp.concatenate(
            [(x * x[:, 4 * g + u:4 * g + u + 1]).astype(jnp.bfloat16)
             for u in range(4)], axis=1)
        wblk = w2_ref[256 * g:256 * (g + 1), :]
        accs[g % 4] = accs[g % 4] + jnp.dot(
            lhs, wblk, preferred_element_type=jnp.float32)
    out_ref[:, :] = (accs[0] + accs[1]) + (accs[2] + accs[3])


def kernel(inputs, W, b):
    x2d = inputs.reshape(_R, _N)
    b2d = b.reshape(1, _M)
    rt = jnp.asarray(_RT_NP)
    out = pl.pallas_call(
        _logic_kernel,
        out_shape=jax.ShapeDtypeStruct((_R, _M), jnp.float32),
        scratch_shapes=[pltpu.VMEM((_N * _N, _M), jnp.bfloat16)],
    )(x2d, W, b2d, rt)
    return out.reshape(_B, _K, _M)
